# baseline (speedup 1.0000x reference)
import functools

import jax
import jax.numpy as jnp
from jax import lax
from jax.experimental import pallas as pl
from jax.experimental.pallas import tpu as pltpu
from jax.experimental.pallas import tpu_sc as plsc

_B = 16384
_DIM = 16
_VOCAB_PAD = 1008
_NC = 2
_NS = 16
_NW = _NC * _NS
_BPW = _B // _NW
_INV = 1.0 / (1.0 + 1e-6)
_BLK = 2048


def _sc_body(t_hbm, uid_hbm, emb_hbm, uid_v, t_v, emb_v):
    wid = lax.axis_index("s") * _NC + lax.axis_index("c")
    base = wid * _BPW

    pltpu.sync_copy(uid_hbm.at[pl.ds(base, _BPW)], uid_v)
    pltpu.sync_copy(t_hbm, t_v)

    iota16 = lax.iota(jnp.int32, 16) * 16

    def body(blk, carry):
        b16 = blk * 16
        u16 = uid_v[pl.ds(b16, 16)]
        src = u16 * 16
        dst = b16 * 16 + iota16
        for f in range(16):
            vals = plsc.load_gather(t_v, [src + f])
            plsc.store_scatter(emb_v, [dst + f], vals)
        return carry

    lax.fori_loop(0, _BPW // 16, body, 0)

    pltpu.sync_copy(emb_v, emb_hbm.at[pl.ds(base * _DIM, _BPW * _DIM)])


@functools.cache
def _sc_gather():
  return pl.kernel(
    _sc_body,
    mesh=plsc.VectorSubcoreMesh(core_axis_name="c", subcore_axis_name="s"),
    compiler_params=pltpu.CompilerParams(needs_layout_passes=False),
    out_type=jax.ShapeDtypeStruct((_B * _DIM,), jnp.float32),
    scratch_types=[
        pltpu.VMEM((_BPW,), jnp.int32),
        pltpu.VMEM((_VOCAB_PAD * _DIM,), jnp.float32),
        pltpu.VMEM((_BPW * _DIM,), jnp.float32),
    ],
  )


def _mlp_body(emb_ref, g_ref, c_ref, a_ref, s_ref, o_ref, v_ref, k_ref,
              w1a_ref, w1s_ref, w1gc_ref, b1_ref, w2_ref, b2_ref, out_ref):
    inv = jnp.float32(_INV)
    pre = jnp.dot(emb_ref[...], w1a_ref[...],
                  preferred_element_type=jnp.float32)
    w1s = w1s_ref[...]
    pre += (a_ref[...] * inv) * w1s[0:1, :]
    pre += (s_ref[...] * inv) * w1s[1:2, :]
    pre += (o_ref[...] * inv) * w1s[2:3, :]
    pre += (v_ref[...] * inv) * w1s[3:4, :]
    pre += (k_ref[...] * inv) * w1s[4:5, :]
    col = lax.broadcasted_iota(jnp.int32, (_BLK, 35), 1)
    sel = (col == g_ref[...]) | (col == c_ref[...] + 3)
    oh = jnp.where(sel, jnp.float32(1.0), jnp.float32(0.0))
    pre += jnp.dot(oh, w1gc_ref[...], preferred_element_type=jnp.float32)
    h = jnp.maximum(pre + b1_ref[...], jnp.float32(0.0))
    out_ref[...] = (
        jnp.dot(h, w2_ref[...], preferred_element_type=jnp.float32)
        + b2_ref[...]
    )


def _mlp(emb, g, c, a, s, o, v, k, w1a, w1s, w1gc, b1, w2, b2):
    row_spec = pl.BlockSpec((_BLK, 1), lambda i: (i, 0))
    full = lambda shape: pl.BlockSpec(shape, lambda i: (0, 0))
    return pl.pallas_call(
        _mlp_body,
        grid=(_B // _BLK,),
        in_specs=[
            pl.BlockSpec((_BLK, _DIM), lambda i: (i, 0)),
            row_spec, row_spec, row_spec, row_spec, row_spec, row_spec,
            row_spec,
            full((_DIM, _DIM)), full((5, _DIM)), full((35, _DIM)),
            full((1, _DIM)), full((_DIM, _DIM)), full((1, _DIM)),
        ],
        out_specs=pl.BlockSpec((_BLK, _DIM), lambda i: (i, 0)),
        out_shape=jax.ShapeDtypeStruct((_B, _DIM), jnp.float32),
    )(emb, g, c, a, s, o, v, k, w1a, w1s, w1gc, b1, w2, b2)


def kernel(user_id, age, sin_month, cos_month, view_count, click_count,
           gender, country, user_table, W1, b1, W2, b2):
    ut = jnp.pad(user_table, ((0, _VOCAB_PAD - user_table.shape[0]), (0, 0)))
    emb = _sc_gather()(ut.reshape(-1), user_id.astype(jnp.int32))
    emb = emb.reshape(_B, _DIM)
    col2 = lambda x: x.reshape(_B, 1)
    return _mlp(
        emb,
        col2(gender.astype(jnp.int32)), col2(country.astype(jnp.int32)),
        col2(age), col2(sin_month), col2(cos_month),
        col2(view_count), col2(click_count),
        W1[:_DIM], W1[_DIM:_DIM + 5], W1[_DIM + 5:],
        b1.reshape(1, _DIM), W2, b2.reshape(1, _DIM),
    )


# speedup vs baseline: 1.5701x; 1.5701x over previous
import functools

import jax
import jax.numpy as jnp
from jax import lax
from jax.experimental import pallas as pl
from jax.experimental.pallas import tpu as pltpu
from jax.experimental.pallas import tpu_sc as plsc

_B = 16384
_DIM = 16
_VOCAB_PAD = 1008
_NC = 2
_NS = 16
_NW = _NC * _NS
_BPW = _B // _NW
_INV = 1.0 / (1.0 + 1e-6)
_BLK = 2048


def _sc_body(t_hbm, w1r_hbm, sp_hbm, uid_hbm, g_hbm, c_hbm,
             a_hbm, s_hbm, o_hbm, v_hbm, k_hbm,
             embt_hbm, restt_hbm,
             uid_v, g_v, c_v, a_v, s_v, o_v, v_v, k_v,
             t_v, w1r_v, sp_v, embt_v, restt_v):
    wid = lax.axis_index("s") * _NC + lax.axis_index("c")
    base = wid * _BPW

    pltpu.sync_copy(uid_hbm.at[pl.ds(base, _BPW)], uid_v)
    pltpu.sync_copy(g_hbm.at[pl.ds(base, _BPW)], g_v)
    pltpu.sync_copy(c_hbm.at[pl.ds(base, _BPW)], c_v)
    pltpu.sync_copy(a_hbm.at[pl.ds(base, _BPW)], a_v)
    pltpu.sync_copy(s_hbm.at[pl.ds(base, _BPW)], s_v)
    pltpu.sync_copy(o_hbm.at[pl.ds(base, _BPW)], o_v)
    pltpu.sync_copy(v_hbm.at[pl.ds(base, _BPW)], v_v)
    pltpu.sync_copy(k_hbm.at[pl.ds(base, _BPW)], k_v)
    pltpu.sync_copy(t_hbm, t_v)
    pltpu.sync_copy(w1r_hbm, w1r_v)
    pltpu.sync_copy(sp_hbm, sp_v)

    inv = jnp.float32(_INV)

    def body(blk, carry):
        b16 = blk * 16
        u16 = uid_v[pl.ds(b16, 16)]
        g16 = g_v[pl.ds(b16, 16)]
        c16 = c_v[pl.ds(b16, 16)]
        a16 = a_v[pl.ds(b16, 16)] * inv
        s16 = s_v[pl.ds(b16, 16)] * inv
        o16 = o_v[pl.ds(b16, 16)] * inv
        v16 = v_v[pl.ds(b16, 16)] * inv
        k16 = k_v[pl.ds(b16, 16)] * inv
        src = u16 * 16
        gidx = g16 * 16 + 80
        cidx = c16 * 16 + 128
        for f in range(16):
            embt_v[f, pl.ds(b16, 16)] = plsc.load_gather(t_v, [src + f])
            acc = (plsc.load_gather(w1r_v, [gidx + f])
                   + plsc.load_gather(w1r_v, [cidx + f]))
            acc = acc + a16 * sp_v[f, :]
            acc = acc + s16 * sp_v[16 + f, :]
            acc = acc + o16 * sp_v[32 + f, :]
            acc = acc + v16 * sp_v[48 + f, :]
            acc = acc + k16 * sp_v[64 + f, :]
            restt_v[f, pl.ds(b16, 16)] = acc
        return carry

    lax.fori_loop(0, _BPW // 16, body, 0)

    pltpu.sync_copy(embt_v, embt_hbm.at[:, pl.ds(base, _BPW)])
    pltpu.sync_copy(restt_v, restt_hbm.at[:, pl.ds(base, _BPW)])


@functools.cache
def _sc_stage():
  return pl.kernel(
    _sc_body,
    mesh=plsc.VectorSubcoreMesh(core_axis_name="c", subcore_axis_name="s"),
    compiler_params=pltpu.CompilerParams(needs_layout_passes=False),
    out_type=[
        jax.ShapeDtypeStruct((_DIM, _B), jnp.float32),
        jax.ShapeDtypeStruct((_DIM, _B), jnp.float32),
    ],
    scratch_types=[
        pltpu.VMEM((_BPW,), jnp.int32),
        pltpu.VMEM((_BPW,), jnp.int32),
        pltpu.VMEM((_BPW,), jnp.int32),
        pltpu.VMEM((_BPW,), jnp.float32),
        pltpu.VMEM((_BPW,), jnp.float32),
        pltpu.VMEM((_BPW,), jnp.float32),
        pltpu.VMEM((_BPW,), jnp.float32),
        pltpu.VMEM((_BPW,), jnp.float32),
        pltpu.VMEM((_VOCAB_PAD * _DIM,), jnp.float32),
        pltpu.VMEM((40 * _DIM,), jnp.float32),
        pltpu.VMEM((80, _DIM), jnp.float32),
        pltpu.VMEM((_DIM, _BPW), jnp.float32),
        pltpu.VMEM((_DIM, _BPW), jnp.float32),
    ],
  )


def _mlp_body(embt_ref, restt_ref, w1a_ref, b1_ref, w2_ref, b2_ref, out_ref):
    cdim = (((0,), (0,)), ((), ()))
    pret = lax.dot_general(w1a_ref[...], embt_ref[...], cdim,
                           preferred_element_type=jnp.float32)
    pret = pret + restt_ref[...] + b1_ref[...]
    ht = jnp.maximum(pret, jnp.float32(0.0))
    outt = lax.dot_general(w2_ref[...], ht, cdim,
                           preferred_element_type=jnp.float32)
    out_ref[...] = (outt + b2_ref[...]).T


def _mlp(embt, restt, w1a, b1, w2, b2):
    tcol_spec = pl.BlockSpec((_DIM, _BLK), lambda i: (0, i))
    full = lambda shape: pl.BlockSpec(shape, lambda i: (0, 0))
    return pl.pallas_call(
        _mlp_body,
        grid=(_B // _BLK,),
        in_specs=[
            tcol_spec, tcol_spec,
            full((_DIM, _DIM)), full((_DIM, 1)),
            full((_DIM, _DIM)), full((_DIM, 1)),
        ],
        out_specs=pl.BlockSpec((_BLK, _DIM), lambda i: (i, 0)),
        out_shape=jax.ShapeDtypeStruct((_B, _DIM), jnp.float32),
    )(embt, restt, w1a, b1, w2, b2)


def kernel(user_id, age, sin_month, cos_month, view_count, click_count,
           gender, country, user_table, W1, b1, W2, b2):
    ut = jnp.pad(user_table, ((0, _VOCAB_PAD - user_table.shape[0]), (0, 0)))
    sp = jnp.broadcast_to(W1[_DIM:_DIM + 5].reshape(80, 1), (80, _DIM))
    embt, restt = _sc_stage()(
        ut.reshape(-1), W1[_DIM:].reshape(-1), sp,
        user_id.astype(jnp.int32), gender.astype(jnp.int32),
        country.astype(jnp.int32),
        age, sin_month, cos_month, view_count, click_count,
    )
    return _mlp(embt, restt, W1[:_DIM], b1.reshape(_DIM, 1),
                W2, b2.reshape(_DIM, 1))
